# baseline (speedup 1.0000x reference)
import functools

import jax
import jax.numpy as jnp
from jax import lax
from jax.experimental import pallas as pl
from jax.experimental.pallas import tpu as pltpu
from jax.experimental.pallas import tpu_sc as plsc

N_NODES = 10000
N_PAD = 10240
N_EDGES = 320000
CH = 128
NW = 32
EPW = N_EDGES // NW
G = 80
NCH = EPW // G
EPS = N_EDGES // 16
NCS = EPS // G
RING = 4
LEAD = 3
NTAIL = NCS - (NCS // RING) * RING


@functools.cache
def _sc_kernels():
    mesh = plsc.VectorSubcoreMesh(core_axis_name="c", subcore_axis_name="s")

    @functools.partial(
        pl.kernel,
        mesh=mesh,
        out_type=jax.ShapeDtypeStruct((2 * N_PAD,), jnp.float32),
        scratch_types=[
            pltpu.VMEM((NCH, G), jnp.int32),
            pltpu.VMEM((NCH, G), jnp.float32),
            pltpu.VMEM((N_PAD // 16,), jnp.float32),
            pltpu.VMEM_SHARED((N_PAD,), jnp.float32),
        ],
    )
    def deg_kernel(dst_hbm, ew_hbm, out_hbm, dst_v, ew_v, zb_v, deg_sh):
        cid = lax.axis_index("c")
        sid = lax.axis_index("s")
        wid = sid * 2 + cid
        seg = N_PAD // 16

        def zloop(i, _):
            zb_v[pl.ds(i * 16, 16)] = jnp.zeros((16,), jnp.float32)
            return 0
        lax.fori_loop(0, seg // 16, zloop, 0)
        pltpu.sync_copy(zb_v, deg_sh.at[pl.ds(sid * seg, seg)])
        plsc.subcore_barrier()

        pltpu.sync_copy(dst_hbm.at[wid], dst_v)
        pltpu.sync_copy(ew_hbm.at[wid], ew_v)

        def body(c, _):
            pltpu.sync_copy(ew_v.at[c], deg_sh.at[dst_v.at[c]], add=True)
            return 0
        lax.fori_loop(0, NCH, body, 0)
        plsc.subcore_barrier()

        pltpu.sync_copy(deg_sh.at[pl.ds(sid * seg, seg)],
                        out_hbm.at[pl.ds(cid * N_PAD + sid * seg, seg)])

    @functools.partial(
        pl.kernel,
        mesh=mesh,
        compiler_params=pltpu.CompilerParams(use_tc_tiling_on_sc=False),
        out_type=jax.ShapeDtypeStruct((2, N_PAD, CH // 2), jnp.float32),
        scratch_types=[
            pltpu.VMEM((EPS,), jnp.int32),
            pltpu.VMEM((NCS, G), jnp.int32),
            pltpu.VMEM((EPS,), jnp.float32),
            pltpu.VMEM((RING, G, CH // 2), jnp.float32),
            pltpu.VMEM((128, CH // 2), jnp.float32),
            pltpu.VMEM_SHARED((N_PAD, CH // 2), jnp.float32),
        ] + [pltpu.SemaphoreType.DMA] * (2 * RING),
    )
    def agg_kernel(src2_hbm, dst_hbm, ew_hbm, zv_hbm, out_hbm,
                   src_v, dst_v, ew_v, bufs, zb_v, agg_sh, *sems):
        gsem = sems[:RING]
        ssem = sems[RING:]
        cid = lax.axis_index("c")
        sid = lax.axis_index("s")
        rows = N_PAD // 16

        def zloop(i, _):
            zb_v[i // 4, pl.ds((i % 4) * 16, 16)] = jnp.zeros((16,), jnp.float32)
            return 0
        lax.fori_loop(0, 128 * 4, zloop, 0)
        for t in range(5):
            pltpu.sync_copy(zb_v, agg_sh.at[pl.ds(sid * rows + t * 128, 128)])
        plsc.subcore_barrier()

        pltpu.sync_copy(src2_hbm.at[sid], src_v)
        pltpu.sync_copy(dst_hbm.at[sid], dst_v)
        pltpu.sync_copy(ew_hbm.at[sid], ew_v)

        def adj(i, _):
            src_v[pl.ds(i * 16, 16)] = src_v[pl.ds(i * 16, 16)] + cid
            return 0
        lax.fori_loop(0, EPS // 16, adj, 0)

        def scale(buf, cc):
            base = cc * G

            def grouploop(q, _):
                wvec = ew_v[pl.ds(base + q * 16, 16)]
                for r in range(16):
                    w = lax.gather(
                        wvec, jnp.full((16, 1), r, jnp.int32),
                        lax.GatherDimensionNumbers(
                            offset_dims=(), collapsed_slice_dims=(0,),
                            start_index_map=(0,)),
                        (1,), mode=lax.GatherScatterMode.PROMISE_IN_BOUNDS)
                    j = q * 16 + r
                    for k in range(CH // 32):
                        buf[j, pl.ds(k * 16, 16)] = (
                            buf[j, pl.ds(k * 16, 16)] * w)
                return 0
            lax.fori_loop(0, G // 16, grouploop, 0)

        def gidx(c):
            return src_v.at[pl.ds(c * G, G)]

        for b in range(LEAD):
            pltpu.make_async_copy(zv_hbm.at[gidx(b)], bufs.at[b], gsem[b]).start()

        def body(i, _):
            for b in range(RING):
                c = RING * i + b
                nb = (b + LEAD) % RING
                pltpu.make_async_copy(
                    zv_hbm.at[gidx(c)], bufs.at[b], gsem[b]).wait()
                scale(bufs.at[b], c)
                pltpu.sync_copy(bufs.at[b], agg_sh.at[dst_v.at[c]], add=True)

                @pl.when(c + LEAD < NCS)
                def _():
                    pltpu.make_async_copy(
                        zv_hbm.at[gidx(c + LEAD)], bufs.at[nb],
                        gsem[nb]).start()
            return 0
        lax.fori_loop(0, NCS // RING, body, 0)
        for b in range(NTAIL):
            c = (NCS // RING) * RING + b
            pltpu.make_async_copy(zv_hbm.at[gidx(c)], bufs.at[b], gsem[b]).wait()
            scale(bufs.at[b], c)
            pltpu.sync_copy(bufs.at[b], agg_sh.at[dst_v.at[c]], add=True)
        plsc.subcore_barrier()

        pltpu.sync_copy(agg_sh.at[pl.ds(sid * rows, rows)],
                        out_hbm.at[cid, pl.ds(sid * rows, rows)])

    return deg_kernel, agg_kernel


def _scale_body(degp_ref, x_ref, Wz_ref, bz_ref, Wlz_ref, blz_ref,
                Wh_ref, bh_ref, Wlh_ref, blh_ref,
                z_ref, dis_ref, WzF_ref, blzF_ref, WhF_ref, blhF_ref):
    deg = degp_ref[0, :N_NODES] + degp_ref[1, :N_NODES] + 1.0
    dis = jnp.where(deg > 0, lax.rsqrt(deg), 0.0)
    z_ref[...] = dis[:, None] * x_ref[...]
    dis_ref[...] = dis[:, None]
    WzF_ref[...] = jnp.dot(Wz_ref[...], Wlz_ref[...],
                           preferred_element_type=jnp.float32)
    blzF_ref[...] = jnp.dot(bz_ref[...], Wlz_ref[...],
                            preferred_element_type=jnp.float32) + blz_ref[...]
    WhF_ref[...] = jnp.dot(Wh_ref[...], Wlh_ref[...],
                           preferred_element_type=jnp.float32)
    blhF_ref[...] = jnp.dot(bh_ref[...], Wlh_ref[...],
                            preferred_element_type=jnp.float32) + blh_ref[...]


def _tc_scale(degp, x, Wz, bz, Wlz, blz, Wh, bh, Wlh, blh):
    return pl.pallas_call(
        _scale_body,
        out_shape=(
            jax.ShapeDtypeStruct((N_NODES, CH), jnp.float32),
            jax.ShapeDtypeStruct((N_NODES, 1), jnp.float32),
            jax.ShapeDtypeStruct((CH, CH), jnp.float32),
            jax.ShapeDtypeStruct((1, CH), jnp.float32),
            jax.ShapeDtypeStruct((CH, CH), jnp.float32),
            jax.ShapeDtypeStruct((1, CH), jnp.float32),
        ),
    )(degp, x, Wz, bz, Wlz, blz, Wh, bh, Wlh, blh)


_RB = 1000


def _dense_body(dis_ref, sp_ref, z_ref, Wz_ref, blz_ref, Wh_ref, blh_ref,
                Wo_ref, bo_ref, out_ref):
    s = jnp.concatenate([sp_ref[0], sp_ref[1]], axis=1)
    agg = dis_ref[...] * (s + z_ref[...])
    gz = jnp.dot(agg, Wz_ref[...], preferred_element_type=jnp.float32)
    zg = jax.nn.sigmoid(gz + blz_ref[...])
    gh = jnp.dot(agg, Wh_ref[...], preferred_element_type=jnp.float32)
    ht = jnp.tanh(gh + blh_ref[...])
    h = jax.nn.relu((1.0 - zg) * ht)
    out_ref[...] = (
        jnp.dot(h, Wo_ref[...], preferred_element_type=jnp.float32)
        + bo_ref[...])


def _tc_dense(dis, sp, z, Wz, blz, Wh, blh, Wo, bo):
    nblk = N_NODES // _RB
    return pl.pallas_call(
        _dense_body,
        grid=(nblk,),
        in_specs=[
            pl.BlockSpec((_RB, 1), lambda i: (i, 0)),
            pl.BlockSpec((2, _RB, CH // 2), lambda i: (0, i, 0)),
            pl.BlockSpec((_RB, CH), lambda i: (i, 0)),
            pl.BlockSpec((CH, CH), lambda i: (0, 0)),
            pl.BlockSpec((1, CH), lambda i: (0, 0)),
            pl.BlockSpec((CH, CH), lambda i: (0, 0)),
            pl.BlockSpec((1, CH), lambda i: (0, 0)),
            pl.BlockSpec((CH, 32), lambda i: (0, 0)),
            pl.BlockSpec((1, 32), lambda i: (0, 0)),
        ],
        out_specs=pl.BlockSpec((_RB, 32), lambda i: (i, 0)),
        out_shape=jax.ShapeDtypeStruct((N_NODES, 32), jnp.float32),
    )(dis, sp, z, Wz, blz, Wh, blh, Wo, bo)


def kernel(x, edge_index, edge_weight, W_z, b_z, W_r, b_r, W_h, b_h,
           Wl_z, bl_z, Wl_r, bl_r, Wl_h, bl_h, W_out, b_out):
    src_i = edge_index[0].astype(jnp.int32)
    dst_i = edge_index[1].astype(jnp.int32)
    ew_f = edge_weight.astype(jnp.float32)

    deg_kernel, agg_kernel = _sc_kernels()
    degp = deg_kernel(dst_i.reshape(NW, NCH, G),
                      ew_f.reshape(NW, NCH, G)).reshape(2, N_PAD)
    z, dis, WzF, blzF, WhF, blhF = _tc_scale(
        degp, x, W_z, b_z.reshape(1, CH), Wl_z[:CH], bl_z.reshape(1, CH),
        W_h, b_h.reshape(1, CH), Wl_h[:CH], bl_h.reshape(1, CH))
    sp = agg_kernel((src_i * 2).reshape(16, EPS),
                    dst_i.reshape(16, NCS, G),
                    ew_f.reshape(16, EPS),
                    z.reshape(2 * N_NODES, CH // 2))

    return _tc_dense(dis, sp, z, WzF, blzF, WhF, blhF,
                     W_out, b_out.reshape(1, 32))


# speedup vs baseline: 1.7307x; 1.7307x over previous
import functools

import jax
import jax.numpy as jnp
from jax import lax
from jax.experimental import pallas as pl
from jax.experimental.pallas import tpu as pltpu
from jax.experimental.pallas import tpu_sc as plsc

N_NODES = 10000
N_PAD = 10240
N_EDGES = 320000
CH = 128
NW = 32
EPW = N_EDGES // NW
G = 80
NCH = EPW // G
EPS = N_EDGES // 16
NCS = EPS // G
RING = 4
LEAD = 3
NTAIL = NCS - (NCS // RING) * RING


@functools.cache
def _sc_kernels():
    mesh = plsc.VectorSubcoreMesh(core_axis_name="c", subcore_axis_name="s")

    @functools.partial(
        pl.kernel,
        mesh=mesh,
        out_type=jax.ShapeDtypeStruct((2 * N_PAD,), jnp.float32),
        scratch_types=[
            pltpu.VMEM((NCH, G), jnp.int32),
            pltpu.VMEM((NCH, G), jnp.float32),
            pltpu.VMEM((N_PAD // 16,), jnp.float32),
            pltpu.VMEM_SHARED((N_PAD,), jnp.float32),
        ],
    )
    def deg_kernel(dst_hbm, ew_hbm, out_hbm, dst_v, ew_v, zb_v, deg_sh):
        cid = lax.axis_index("c")
        sid = lax.axis_index("s")
        wid = sid * 2 + cid
        seg = N_PAD // 16

        def zloop(i, _):
            zb_v[pl.ds(i * 16, 16)] = jnp.zeros((16,), jnp.float32)
            return 0
        lax.fori_loop(0, seg // 16, zloop, 0)
        pltpu.sync_copy(zb_v, deg_sh.at[pl.ds(sid * seg, seg)])
        plsc.subcore_barrier()

        pltpu.sync_copy(dst_hbm.at[wid], dst_v)
        pltpu.sync_copy(ew_hbm.at[wid], ew_v)

        def body(c, _):
            pltpu.sync_copy(ew_v.at[c], deg_sh.at[dst_v.at[c]], add=True)
            return 0
        lax.fori_loop(0, NCH, body, 0)
        plsc.subcore_barrier()

        pltpu.sync_copy(deg_sh.at[pl.ds(sid * seg, seg)],
                        out_hbm.at[pl.ds(cid * N_PAD + sid * seg, seg)])

    @functools.partial(
        pl.kernel,
        mesh=mesh,
        compiler_params=pltpu.CompilerParams(use_tc_tiling_on_sc=False),
        out_type=jax.ShapeDtypeStruct((2, N_PAD, CH // 2), jnp.float32),
        scratch_types=[
            pltpu.VMEM((EPS,), jnp.int32),
            pltpu.VMEM((NCS, G), jnp.int32),
            pltpu.VMEM((EPS,), jnp.float32),
            pltpu.VMEM((RING, G, CH // 2), jnp.float32),
            pltpu.VMEM((128, CH // 2), jnp.float32),
            pltpu.VMEM_SHARED((N_PAD, CH // 2), jnp.float32),
        ] + [pltpu.SemaphoreType.DMA] * (2 * RING),
    )
    def agg_kernel(src2_hbm, dst_hbm, ew_hbm, zv_hbm, out_hbm,
                   src_v, dst_v, ew_v, bufs, zb_v, agg_sh, *sems):
        gsem = sems[:RING]
        ssem = sems[RING:]
        cid = lax.axis_index("c")
        sid = lax.axis_index("s")
        rows = N_PAD // 16

        def zloop(i, _):
            zb_v[i // 4, pl.ds((i % 4) * 16, 16)] = jnp.zeros((16,), jnp.float32)
            return 0
        lax.fori_loop(0, 128 * 4, zloop, 0)
        for t in range(5):
            pltpu.sync_copy(zb_v, agg_sh.at[pl.ds(sid * rows + t * 128, 128)])
        plsc.subcore_barrier()

        pltpu.sync_copy(src2_hbm.at[sid], src_v)
        pltpu.sync_copy(dst_hbm.at[sid], dst_v)
        pltpu.sync_copy(ew_hbm.at[sid], ew_v)

        def adj(i, _):
            src_v[pl.ds(i * 16, 16)] = src_v[pl.ds(i * 16, 16)] + cid
            return 0
        lax.fori_loop(0, EPS // 16, adj, 0)

        _dn = lax.GatherDimensionNumbers(
            offset_dims=(), collapsed_slice_dims=(0,), start_index_map=(0,))

        def scale(buf, cc):
            base = cc * G
            for q in range(G // 16):
                wvec = ew_v[pl.ds(base + q * 16, 16)]
                ws = [lax.gather(wvec, jnp.full((16, 1), r, jnp.int32), _dn,
                                 (1,),
                                 mode=lax.GatherScatterMode.PROMISE_IN_BOUNDS)
                      for r in range(16)]
                for r in range(16):
                    j = q * 16 + r
                    for k in range(CH // 32):
                        buf[j, pl.ds(k * 16, 16)] = (
                            buf[j, pl.ds(k * 16, 16)] * ws[r])

        def gidx(c):
            return src_v.at[pl.ds(c * G, G)]

        for b in range(LEAD):
            pltpu.make_async_copy(zv_hbm.at[gidx(b)], bufs.at[b], gsem[b]).start()

        def body(i, _):
            for b in range(RING):
                c = RING * i + b
                nb = (b + LEAD) % RING
                pltpu.make_async_copy(
                    zv_hbm.at[gidx(c)], bufs.at[b], gsem[b]).wait()
                scale(bufs.at[b], c)
                pltpu.sync_copy(bufs.at[b], agg_sh.at[dst_v.at[c]], add=True)

                @pl.when(c + LEAD < NCS)
                def _():
                    pltpu.make_async_copy(
                        zv_hbm.at[gidx(c + LEAD)], bufs.at[nb],
                        gsem[nb]).start()
            return 0
        lax.fori_loop(0, NCS // RING, body, 0)
        for b in range(NTAIL):
            c = (NCS // RING) * RING + b
            pltpu.make_async_copy(zv_hbm.at[gidx(c)], bufs.at[b], gsem[b]).wait()
            scale(bufs.at[b], c)
            pltpu.sync_copy(bufs.at[b], agg_sh.at[dst_v.at[c]], add=True)
        plsc.subcore_barrier()

        pltpu.sync_copy(agg_sh.at[pl.ds(sid * rows, rows)],
                        out_hbm.at[cid, pl.ds(sid * rows, rows)])

    return deg_kernel, agg_kernel


def _scale_body(degp_ref, x_ref, Wz_ref, bz_ref, Wlz_ref, blz_ref,
                Wh_ref, bh_ref, Wlh_ref, blh_ref,
                z_ref, dis_ref, WzF_ref, blzF_ref, WhF_ref, blhF_ref):
    deg = degp_ref[0, :N_NODES] + degp_ref[1, :N_NODES] + 1.0
    dis = jnp.where(deg > 0, lax.rsqrt(deg), 0.0)
    z_ref[...] = dis[:, None] * x_ref[...]
    dis_ref[...] = dis[:, None]
    WzF_ref[...] = jnp.dot(Wz_ref[...], Wlz_ref[...],
                           preferred_element_type=jnp.float32)
    blzF_ref[...] = jnp.dot(bz_ref[...], Wlz_ref[...],
                            preferred_element_type=jnp.float32) + blz_ref[...]
    WhF_ref[...] = jnp.dot(Wh_ref[...], Wlh_ref[...],
                           preferred_element_type=jnp.float32)
    blhF_ref[...] = jnp.dot(bh_ref[...], Wlh_ref[...],
                            preferred_element_type=jnp.float32) + blh_ref[...]


def _tc_scale(degp, x, Wz, bz, Wlz, blz, Wh, bh, Wlh, blh):
    return pl.pallas_call(
        _scale_body,
        out_shape=(
            jax.ShapeDtypeStruct((N_NODES, CH), jnp.float32),
            jax.ShapeDtypeStruct((N_NODES, 1), jnp.float32),
            jax.ShapeDtypeStruct((CH, CH), jnp.float32),
            jax.ShapeDtypeStruct((1, CH), jnp.float32),
            jax.ShapeDtypeStruct((CH, CH), jnp.float32),
            jax.ShapeDtypeStruct((1, CH), jnp.float32),
        ),
    )(degp, x, Wz, bz, Wlz, blz, Wh, bh, Wlh, blh)


_RB = 1000


def _dense_body(dis_ref, sp_ref, z_ref, Wz_ref, blz_ref, Wh_ref, blh_ref,
                Wo_ref, bo_ref, out_ref):
    s = jnp.concatenate([sp_ref[0], sp_ref[1]], axis=1)
    agg = dis_ref[...] * (s + z_ref[...])
    gz = jnp.dot(agg, Wz_ref[...], preferred_element_type=jnp.float32)
    zg = jax.nn.sigmoid(gz + blz_ref[...])
    gh = jnp.dot(agg, Wh_ref[...], preferred_element_type=jnp.float32)
    ht = jnp.tanh(gh + blh_ref[...])
    h = jax.nn.relu((1.0 - zg) * ht)
    out_ref[...] = (
        jnp.dot(h, Wo_ref[...], preferred_element_type=jnp.float32)
        + bo_ref[...])


def _tc_dense(dis, sp, z, Wz, blz, Wh, blh, Wo, bo):
    nblk = N_NODES // _RB
    return pl.pallas_call(
        _dense_body,
        grid=(nblk,),
        in_specs=[
            pl.BlockSpec((_RB, 1), lambda i: (i, 0)),
            pl.BlockSpec((2, _RB, CH // 2), lambda i: (0, i, 0)),
            pl.BlockSpec((_RB, CH), lambda i: (i, 0)),
            pl.BlockSpec((CH, CH), lambda i: (0, 0)),
            pl.BlockSpec((1, CH), lambda i: (0, 0)),
            pl.BlockSpec((CH, CH), lambda i: (0, 0)),
            pl.BlockSpec((1, CH), lambda i: (0, 0)),
            pl.BlockSpec((CH, 32), lambda i: (0, 0)),
            pl.BlockSpec((1, 32), lambda i: (0, 0)),
        ],
        out_specs=pl.BlockSpec((_RB, 32), lambda i: (i, 0)),
        out_shape=jax.ShapeDtypeStruct((N_NODES, 32), jnp.float32),
    )(dis, sp, z, Wz, blz, Wh, blh, Wo, bo)


def kernel(x, edge_index, edge_weight, W_z, b_z, W_r, b_r, W_h, b_h,
           Wl_z, bl_z, Wl_r, bl_r, Wl_h, bl_h, W_out, b_out):
    src_i = edge_index[0].astype(jnp.int32)
    dst_i = edge_index[1].astype(jnp.int32)
    ew_f = edge_weight.astype(jnp.float32)

    deg_kernel, agg_kernel = _sc_kernels()
    degp = deg_kernel(dst_i.reshape(NW, NCH, G),
                      ew_f.reshape(NW, NCH, G)).reshape(2, N_PAD)
    z, dis, WzF, blzF, WhF, blhF = _tc_scale(
        degp, x, W_z, b_z.reshape(1, CH), Wl_z[:CH], bl_z.reshape(1, CH),
        W_h, b_h.reshape(1, CH), Wl_h[:CH], bl_h.reshape(1, CH))
    sp = agg_kernel((src_i * 2).reshape(16, EPS),
                    dst_i.reshape(16, NCS, G),
                    ew_f.reshape(16, EPS),
                    z.reshape(2 * N_NODES, CH // 2))

    return _tc_dense(dis, sp, z, WzF, blzF, WhF, blhF,
                     W_out, b_out.reshape(1, 32))


# speedup vs baseline: 1.9949x; 1.1526x over previous
import functools

import jax
import jax.numpy as jnp
from jax import lax
from jax.experimental import pallas as pl
from jax.experimental.pallas import tpu as pltpu
from jax.experimental.pallas import tpu_sc as plsc

N_NODES = 10000
N_PAD = 10240
N_EDGES = 320000
CH = 128
NW = 32
EPW = N_EDGES // NW
G = 80
NCH = EPW // G
EPS = N_EDGES // 16
NCS = EPS // G
RING = 4
LEAD = 3
NTAIL = NCS - (NCS // RING) * RING


@functools.cache
def _sc_kernels():
    mesh = plsc.VectorSubcoreMesh(core_axis_name="c", subcore_axis_name="s")

    @functools.partial(
        pl.kernel,
        mesh=mesh,
        out_type=jax.ShapeDtypeStruct((2 * N_PAD,), jnp.float32),
        scratch_types=[
            pltpu.VMEM((NCH, G), jnp.int32),
            pltpu.VMEM((NCH, G), jnp.float32),
            pltpu.VMEM((N_PAD // 16,), jnp.float32),
            pltpu.VMEM_SHARED((N_PAD,), jnp.float32),
        ],
    )
    def deg_kernel(dst_hbm, ew_hbm, out_hbm, dst_v, ew_v, zb_v, deg_sh):
        cid = lax.axis_index("c")
        sid = lax.axis_index("s")
        wid = sid * 2 + cid
        seg = N_PAD // 16

        def zloop(i, _):
            zb_v[pl.ds(i * 16, 16)] = jnp.zeros((16,), jnp.float32)
            return 0
        lax.fori_loop(0, seg // 16, zloop, 0)
        pltpu.sync_copy(zb_v, deg_sh.at[pl.ds(sid * seg, seg)])
        plsc.subcore_barrier()

        pltpu.sync_copy(dst_hbm.at[wid], dst_v)
        pltpu.sync_copy(ew_hbm.at[wid], ew_v)

        def body(c, _):
            pltpu.sync_copy(ew_v.at[c], deg_sh.at[dst_v.at[c]], add=True)
            return 0
        lax.fori_loop(0, NCH, body, 0)
        plsc.subcore_barrier()

        pltpu.sync_copy(deg_sh.at[pl.ds(sid * seg, seg)],
                        out_hbm.at[pl.ds(cid * N_PAD + sid * seg, seg)])

    @functools.partial(
        pl.kernel,
        mesh=mesh,
        compiler_params=pltpu.CompilerParams(use_tc_tiling_on_sc=False),
        out_type=jax.ShapeDtypeStruct((2, N_PAD, CH // 2), jnp.float32),
        scratch_types=[
            pltpu.VMEM((EPS,), jnp.int32),
            pltpu.VMEM((NCS, G), jnp.int32),
            pltpu.VMEM((EPS,), jnp.float32),
            pltpu.VMEM((RING, G, CH // 2), jnp.float32),
            pltpu.VMEM((128, CH // 2), jnp.float32),
            pltpu.VMEM_SHARED((N_PAD, CH // 2), jnp.float32),
        ] + [pltpu.SemaphoreType.DMA] * (2 * RING),
    )
    def agg_kernel(src2_hbm, dst_hbm, ew_hbm, zv_hbm, out_hbm,
                   src_v, dst_v, ew_v, bufs, zb_v, agg_sh, *sems):
        gsem = sems[:RING]
        ssem = sems[RING:]
        cid = lax.axis_index("c")
        sid = lax.axis_index("s")
        rows = N_PAD // 16

        def zloop(i, _):
            zb_v[i // 4, pl.ds((i % 4) * 16, 16)] = jnp.zeros((16,), jnp.float32)
            return 0
        lax.fori_loop(0, 128 * 4, zloop, 0)
        for t in range(5):
            pltpu.sync_copy(zb_v, agg_sh.at[pl.ds(sid * rows + t * 128, 128)])
        plsc.subcore_barrier()

        pltpu.sync_copy(src2_hbm.at[sid], src_v)
        pltpu.sync_copy(dst_hbm.at[sid], dst_v)
        pltpu.sync_copy(ew_hbm.at[sid], ew_v)

        def adj(i, _):
            src_v[pl.ds(i * 16, 16)] = src_v[pl.ds(i * 16, 16)] + cid
            return 0
        lax.fori_loop(0, EPS // 16, adj, 0)

        _dn = lax.GatherDimensionNumbers(
            offset_dims=(), collapsed_slice_dims=(0,), start_index_map=(0,))

        def scale(buf, cc):
            base = cc * G
            for q in range(G // 16):
                wvec = ew_v[pl.ds(base + q * 16, 16)]
                ws = [lax.gather(wvec, jnp.full((16, 1), r, jnp.int32), _dn,
                                 (1,),
                                 mode=lax.GatherScatterMode.PROMISE_IN_BOUNDS)
                      for r in range(16)]
                for r in range(16):
                    j = q * 16 + r
                    for k in range(CH // 32):
                        buf[j, pl.ds(k * 16, 16)] = (
                            buf[j, pl.ds(k * 16, 16)] * ws[r])

        def gidx(c):
            return src_v.at[pl.ds(c * G, G)]

        for b in range(LEAD):
            pltpu.make_async_copy(zv_hbm.at[gidx(b)], bufs.at[b], gsem[b]).start()

        def body(i, _):
            for b in range(RING):
                c = RING * i + b
                nb = (b + LEAD) % RING
                pltpu.make_async_copy(
                    zv_hbm.at[gidx(c)], bufs.at[b], gsem[b]).wait()
                scale(bufs.at[b], c)
                pltpu.async_copy(
                    bufs.at[b], agg_sh.at[dst_v.at[c]], ssem[b], add=True)

                @pl.when(c + LEAD < NCS)
                def _():
                    @pl.when(c >= RING - LEAD)
                    def _():
                        pltpu.make_async_copy(
                            bufs.at[nb],
                            agg_sh.at[dst_v.at[c - (RING - LEAD)]],
                            ssem[nb]).wait()
                    pltpu.make_async_copy(
                        zv_hbm.at[gidx(c + LEAD)], bufs.at[nb],
                        gsem[nb]).start()
            return 0
        lax.fori_loop(0, NCS // RING, body, 0)
        nloop = (NCS // RING) * RING
        for b in range(NTAIL):
            c = nloop + b
            pltpu.make_async_copy(zv_hbm.at[gidx(c)], bufs.at[b], gsem[b]).wait()
            scale(bufs.at[b], c)
            pltpu.async_copy(bufs.at[b], agg_sh.at[dst_v.at[c]], ssem[b],
                             add=True)
        for b in range(NTAIL):
            pltpu.make_async_copy(
                bufs.at[b], agg_sh.at[dst_v.at[nloop + b]], ssem[b]).wait()
        for b in range(NTAIL, RING):
            pltpu.make_async_copy(
                bufs.at[b], agg_sh.at[dst_v.at[nloop - RING + b]],
                ssem[b]).wait()
        plsc.subcore_barrier()

        pltpu.sync_copy(agg_sh.at[pl.ds(sid * rows, rows)],
                        out_hbm.at[cid, pl.ds(sid * rows, rows)])

    return deg_kernel, agg_kernel


def _scale_body(degp_ref, x_ref, Wz_ref, bz_ref, Wlz_ref, blz_ref,
                Wh_ref, bh_ref, Wlh_ref, blh_ref,
                z_ref, dis_ref, WzF_ref, blzF_ref, WhF_ref, blhF_ref):
    deg = degp_ref[0, :N_NODES] + degp_ref[1, :N_NODES] + 1.0
    dis = jnp.where(deg > 0, lax.rsqrt(deg), 0.0)
    z_ref[...] = dis[:, None] * x_ref[...]
    dis_ref[...] = dis[:, None]
    WzF_ref[...] = jnp.dot(Wz_ref[...], Wlz_ref[...],
                           preferred_element_type=jnp.float32)
    blzF_ref[...] = jnp.dot(bz_ref[...], Wlz_ref[...],
                            preferred_element_type=jnp.float32) + blz_ref[...]
    WhF_ref[...] = jnp.dot(Wh_ref[...], Wlh_ref[...],
                           preferred_element_type=jnp.float32)
    blhF_ref[...] = jnp.dot(bh_ref[...], Wlh_ref[...],
                            preferred_element_type=jnp.float32) + blh_ref[...]


def _tc_scale(degp, x, Wz, bz, Wlz, blz, Wh, bh, Wlh, blh):
    return pl.pallas_call(
        _scale_body,
        out_shape=(
            jax.ShapeDtypeStruct((N_NODES, CH), jnp.float32),
            jax.ShapeDtypeStruct((N_NODES, 1), jnp.float32),
            jax.ShapeDtypeStruct((CH, CH), jnp.float32),
            jax.ShapeDtypeStruct((1, CH), jnp.float32),
            jax.ShapeDtypeStruct((CH, CH), jnp.float32),
            jax.ShapeDtypeStruct((1, CH), jnp.float32),
        ),
    )(degp, x, Wz, bz, Wlz, blz, Wh, bh, Wlh, blh)


_RB = 1000


def _dense_body(dis_ref, sp_ref, z_ref, Wz_ref, blz_ref, Wh_ref, blh_ref,
                Wo_ref, bo_ref, out_ref):
    s = jnp.concatenate([sp_ref[0], sp_ref[1]], axis=1)
    agg = dis_ref[...] * (s + z_ref[...])
    gz = jnp.dot(agg, Wz_ref[...], preferred_element_type=jnp.float32)
    zg = jax.nn.sigmoid(gz + blz_ref[...])
    gh = jnp.dot(agg, Wh_ref[...], preferred_element_type=jnp.float32)
    ht = jnp.tanh(gh + blh_ref[...])
    h = jax.nn.relu((1.0 - zg) * ht)
    out_ref[...] = (
        jnp.dot(h, Wo_ref[...], preferred_element_type=jnp.float32)
        + bo_ref[...])


def _tc_dense(dis, sp, z, Wz, blz, Wh, blh, Wo, bo):
    nblk = N_NODES // _RB
    return pl.pallas_call(
        _dense_body,
        grid=(nblk,),
        in_specs=[
            pl.BlockSpec((_RB, 1), lambda i: (i, 0)),
            pl.BlockSpec((2, _RB, CH // 2), lambda i: (0, i, 0)),
            pl.BlockSpec((_RB, CH), lambda i: (i, 0)),
            pl.BlockSpec((CH, CH), lambda i: (0, 0)),
            pl.BlockSpec((1, CH), lambda i: (0, 0)),
            pl.BlockSpec((CH, CH), lambda i: (0, 0)),
            pl.BlockSpec((1, CH), lambda i: (0, 0)),
            pl.BlockSpec((CH, 32), lambda i: (0, 0)),
            pl.BlockSpec((1, 32), lambda i: (0, 0)),
        ],
        out_specs=pl.BlockSpec((_RB, 32), lambda i: (i, 0)),
        out_shape=jax.ShapeDtypeStruct((N_NODES, 32), jnp.float32),
    )(dis, sp, z, Wz, blz, Wh, blh, Wo, bo)


def kernel(x, edge_index, edge_weight, W_z, b_z, W_r, b_r, W_h, b_h,
           Wl_z, bl_z, Wl_r, bl_r, Wl_h, bl_h, W_out, b_out):
    src_i = edge_index[0].astype(jnp.int32)
    dst_i = edge_index[1].astype(jnp.int32)
    ew_f = edge_weight.astype(jnp.float32)

    deg_kernel, agg_kernel = _sc_kernels()
    degp = deg_kernel(dst_i.reshape(NW, NCH, G),
                      ew_f.reshape(NW, NCH, G)).reshape(2, N_PAD)
    z, dis, WzF, blzF, WhF, blhF = _tc_scale(
        degp, x, W_z, b_z.reshape(1, CH), Wl_z[:CH], bl_z.reshape(1, CH),
        W_h, b_h.reshape(1, CH), Wl_h[:CH], bl_h.reshape(1, CH))
    sp = agg_kernel((src_i * 2).reshape(16, EPS),
                    dst_i.reshape(16, NCS, G),
                    ew_f.reshape(16, EPS),
                    z.reshape(2 * N_NODES, CH // 2))

    return _tc_dense(dis, sp, z, WzF, blzF, WhF, blhF,
                     W_out, b_out.reshape(1, 32))
